# Initial kernel scaffold; baseline (speedup 1.0000x reference)
import jax
import jax.numpy as jnp
from jax.experimental import pallas as pl


def kernel(x, edge_index, edge_weight, W1, b1, gamma, beta, W2, b2):
    raise NotImplementedError("write your pallas kernel here")


# speedup vs baseline: 2.6013x; 2.6013x over previous
import functools

import jax
import jax.numpy as jnp
from jax import lax
from jax.experimental import pallas as pl
from jax.experimental.pallas import tpu as pltpu
from jax.experimental.pallas import tpu_sc as plsc

_NC = 2
_NS = 16
_LANES = 16
_C = 128
_ZB = 40
_G = 16


def _propagate_sc(h, srcp, dstp, wp):
    nw, ch, c = srcp.shape
    ngrp = ch // _G
    n, f = h.shape
    nblk = n // _ZB
    nvec = f // _LANES
    mesh = plsc.VectorSubcoreMesh(core_axis_name="c", subcore_axis_name="s")

    @functools.partial(
        pl.kernel,
        out_type=jax.ShapeDtypeStruct((_NC, n, f), jnp.float32),
        mesh=mesh,
        scratch_types=[
            pltpu.VMEM((_G, c), jnp.int32),
            pltpu.VMEM((_G, c), jnp.int32),
            pltpu.VMEM((_G, c), jnp.float32),
            pltpu.VMEM((c, f), jnp.float32),
            pltpu.VMEM((_ZB, f), jnp.float32),
            pltpu.VMEM_SHARED((n, f), jnp.float32),
        ],
    )
    def k(h_hbm, src_hbm, dst_hbm, w_hbm, out_hbm, srcv, dstv, wv, rows, buf, acc):
        cid = lax.axis_index("c")
        sid = lax.axis_index("s")
        wid = cid * _NS + sid

        zero = jnp.zeros((_LANES,), jnp.float32)

        @pl.loop(0, _ZB)
        def _(r):
            for q in range(nvec):
                buf[r, pl.ds(q * _LANES, _LANES)] = zero

        @pl.loop(sid, nblk, step=_NS)
        def _(blk):
            pltpu.sync_copy(buf, acc.at[pl.ds(blk * _ZB, _ZB)])

        plsc.subcore_barrier()

        @pl.loop(0, ngrp)
        def _(g):
            gsl = pl.ds(g * _G, _G)
            pltpu.sync_copy(src_hbm.at[wid].at[gsl], srcv)
            pltpu.sync_copy(dst_hbm.at[wid].at[gsl], dstv)
            pltpu.sync_copy(w_hbm.at[wid].at[gsl], wv)

            @pl.loop(0, _G)
            def _(j):
                pltpu.sync_copy(h_hbm.at[srcv.at[j]], rows)

                @pl.loop(0, c, step=_LANES)
                def _(r0):
                    wvec = wv[j, pl.ds(r0, _LANES)]
                    for i in range(_LANES):
                        wsc = wvec[i]
                        for q in range(nvec):
                            sl = pl.ds(q * _LANES, _LANES)
                            rows[r0 + i, sl] = rows[r0 + i, sl] * wsc

                pltpu.sync_copy(rows, acc.at[dstv.at[j]], add=True)

        plsc.subcore_barrier()

        @pl.loop(sid, nblk, step=_NS)
        def _(blk):
            r = blk * _ZB
            pltpu.sync_copy(acc.at[pl.ds(r, _ZB)], buf)
            pltpu.sync_copy(buf, out_hbm.at[cid].at[pl.ds(r, _ZB)])

    return k(h, srcp, dstp, wp)


def _mm_fused(x0, x1, x2, x3, w0, w1, w2, w3, svec, bvec, relu):
    n, f = x0.shape
    fo = w0.shape[1]
    bn = 2000

    def body(x0r, x1r, x2r, x3r, w0r, w1r, w2r, w3r, sr, br, o):
        acc = jnp.dot(x0r[...], w0r[...], preferred_element_type=jnp.float32)
        acc = acc + jnp.dot(x1r[...], w1r[...], preferred_element_type=jnp.float32)
        acc = acc + jnp.dot(x2r[...], w2r[...], preferred_element_type=jnp.float32)
        acc = acc + jnp.dot(x3r[...], w3r[...], preferred_element_type=jnp.float32)
        acc = acc * sr[...] + br[...]
        if relu:
            acc = jnp.maximum(acc, 0.0)
        o[...] = acc

    in_specs = (
        [pl.BlockSpec((bn, f), lambda i: (i, 0))] * 4
        + [pl.BlockSpec((f, fo), lambda i: (0, 0))] * 4
        + [pl.BlockSpec((1, fo), lambda i: (0, 0))] * 2
    )
    return pl.pallas_call(
        body,
        grid=(n // bn,),
        in_specs=in_specs,
        out_specs=pl.BlockSpec((bn, fo), lambda i: (i, 0)),
        out_shape=jax.ShapeDtypeStruct((n, fo), jnp.float32),
    )(x0, x1, x2, x3, w0, w1, w2, w3, svec, bvec)


def kernel(x, edge_index, edge_weight, W1, b1, gamma, beta, W2, b2):
    n, f = x.shape
    e = edge_index.shape[1]
    nw = _NC * _NS
    ch = -(-e // (nw * _C * _G)) * _G
    epad = nw * ch * _C - e

    src = edge_index[0].astype(jnp.int32)
    dst = edge_index[1].astype(jnp.int32)
    w = edge_weight.astype(jnp.float32)
    if epad:
        src = jnp.concatenate([src, jnp.zeros((epad,), jnp.int32)])
        dst = jnp.concatenate([dst, jnp.zeros((epad,), jnp.int32)])
        w = jnp.concatenate([w, jnp.zeros((epad,), jnp.float32)])
    srcp = src.reshape(nw, ch, _C)
    dstp = dst.reshape(nw, ch, _C)
    wp = w.reshape(nw, ch, _C)

    def prop(hcur):
        p = _propagate_sc(hcur, srcp, dstp, wp)
        return p[0] + p[1]

    ax = prop(x)
    a2x = prop(ax)
    a3x = prop(a2x)
    s = gamma * jax.lax.rsqrt(jnp.float32(1.0 + 1e-3))
    bvec = (b1 * s + beta).reshape(1, -1)
    h = _mm_fused(x, ax, a2x, a3x,
                  W1[0:f], W1[f:2 * f], W1[2 * f:3 * f], W1[3 * f:4 * f],
                  s.reshape(1, -1), bvec, relu=True)

    ah = prop(h)
    a2h = prop(ah)
    a3h = prop(a2h)
    fo = W2.shape[1]
    w2p = jnp.pad(W2, ((0, 0), (0, 128 - fo)))
    ones = jnp.ones((1, 128), jnp.float32)
    b2p = jnp.pad(b2, (0, 128 - fo)).reshape(1, -1)
    h2 = h.shape[1]
    z = _mm_fused(h, ah, a2h, a3h,
                  w2p[0:h2], w2p[h2:2 * h2], w2p[2 * h2:3 * h2], w2p[3 * h2:4 * h2],
                  ones, b2p, relu=False)
    return z[:, :fo]


# speedup vs baseline: 6.0787x; 2.3368x over previous
import functools

import jax
import jax.numpy as jnp
from jax import lax
from jax.experimental import pallas as pl
from jax.experimental.pallas import tpu as pltpu
from jax.experimental.pallas import tpu_sc as plsc

_NC = 2
_NS = 16
_LANES = 16
_C = 112
_ZB = 40
_G = 16


def _propagate_sc(h, srcf, dstf, wf, ch):
    n, f = h.shape
    nblk = n // _ZB
    nvec = f // _LANES
    mesh = plsc.VectorSubcoreMesh(core_axis_name="c", subcore_axis_name="s")

    @functools.partial(
        pl.kernel,
        out_type=jax.ShapeDtypeStruct((_NC, n, f), jnp.float32),
        mesh=mesh,
        scratch_types=[
            pltpu.VMEM((6, _C), jnp.int32),
            pltpu.VMEM((6, _C), jnp.int32),
            pltpu.VMEM((6, _C), jnp.float32),
            pltpu.VMEM((_C, f), jnp.float32),
            pltpu.VMEM((_C, f), jnp.float32),
            pltpu.VMEM((_C, f), jnp.float32),
            pltpu.VMEM_SHARED((n, f), jnp.float32),
            pltpu.SemaphoreType.DMA((6,)),
            pltpu.SemaphoreType.DMA((3,)),
            pltpu.SemaphoreType.DMA((3,)),
        ],
    )
    def k(h_hbm, src_hbm, dst_hbm, w_hbm, out_hbm,
          srcb, dstb, wb, rows0, rows1, rows2, acc, esem, gsem, ssem):
        cid = lax.axis_index("c")
        sid = lax.axis_index("s")
        wid = cid * _NS + sid
        base = wid * (ch * _C)
        rows = (rows0, rows1, rows2)

        def estart(j, q):
            sl = pl.ds(base + j * _C, _C)
            pltpu.async_copy(src_hbm.at[sl], srcb.at[q], esem.at[q])
            pltpu.async_copy(dst_hbm.at[sl], dstb.at[q], esem.at[q])
            pltpu.async_copy(w_hbm.at[sl], wb.at[q], esem.at[q])

        def ewait(j, q):
            sl = pl.ds(base + j * _C, _C)
            pltpu.make_async_copy(src_hbm.at[sl], srcb.at[q], esem.at[q]).wait()
            pltpu.make_async_copy(dst_hbm.at[sl], dstb.at[q], esem.at[q]).wait()
            pltpu.make_async_copy(w_hbm.at[sl], wb.at[q], esem.at[q]).wait()

        def gstart(q, b):
            pltpu.async_copy(h_hbm.at[srcb.at[q]], rows[b], gsem.at[b])

        def gwait(q, b):
            pltpu.make_async_copy(h_hbm.at[srcb.at[q]], rows[b], gsem.at[b]).wait()

        def sstart(q, b):
            pltpu.async_copy(rows[b], acc.at[dstb.at[q]], ssem.at[b], add=True)

        def swait(q, b):
            pltpu.make_async_copy(rows[b], acc.at[dstb.at[q]], ssem.at[b]).wait()

        def scale(q, b):
            rb = rows[b]

            @pl.loop(0, _C, step=_LANES)
            def _(r0):
                wvec = wb[q, pl.ds(r0, _LANES)]
                for i in range(_LANES):
                    wsc = wvec[i]
                    for v in range(nvec):
                        sl = pl.ds(v * _LANES, _LANES)
                        rb[r0 + i, sl] = rb[r0 + i, sl] * wsc

        zero = jnp.zeros((_LANES,), jnp.float32)

        @pl.loop(0, _ZB)
        def _(r):
            for v in range(nvec):
                rows0[r, pl.ds(v * _LANES, _LANES)] = zero

        @pl.loop(sid, nblk, step=_NS)
        def _(blk):
            pltpu.sync_copy(rows0.at[pl.ds(0, _ZB)], acc.at[pl.ds(blk * _ZB, _ZB)])

        plsc.subcore_barrier()

        def step(j, jj, first, last_grp):
            b = jj % 3
            if not first or jj > 0:
                swait((jj + 5) % 6, (jj + 2) % 3)
            if not last_grp or jj + 4 < 6:
                estart(j + 4, (jj + 4) % 6)
            if not last_grp or jj + 2 < 6:
                ewait(j + 2, (jj + 2) % 6)
                gstart((jj + 2) % 6, (jj + 2) % 3)
            gwait(jj, b)
            scale(jj, b)
            sstart(jj, b)

        estart(0, 0)
        estart(1, 1)
        estart(2, 2)
        estart(3, 3)
        ewait(0, 0)
        gstart(0, 0)
        ewait(1, 1)
        gstart(1, 1)

        for jj in range(6):
            step(jj, jj, True, ch == 6)

        if ch > 12:
            @pl.loop(1, ch // 6 - 1)
            def _(g):
                j0 = g * 6
                for jj in range(6):
                    step(j0 + jj, jj, False, False)

        if ch > 6:
            j0 = ch - 6
            for jj in range(6):
                step(j0 + jj, jj, False, True)

        swait((ch - 1) % 6, (ch - 1) % 3)

        plsc.subcore_barrier()

        @pl.loop(sid, nblk, step=_NS)
        def _(blk):
            r = blk * _ZB
            pltpu.sync_copy(acc.at[pl.ds(r, _ZB)], rows0.at[pl.ds(0, _ZB)])
            pltpu.sync_copy(rows0.at[pl.ds(0, _ZB)], out_hbm.at[cid].at[pl.ds(r, _ZB)])

    return k(h, srcf, dstf, wf)


def _mm_fused(x0, x1, x2, x3, w0, w1, w2, w3, svec, bvec, relu):
    n, f = x0.shape
    fo = w0.shape[1]
    bn = 2000

    def body(x0r, x1r, x2r, x3r, w0r, w1r, w2r, w3r, sr, br, o):
        acc = jnp.dot(x0r[...], w0r[...], preferred_element_type=jnp.float32)
        acc = acc + jnp.dot(x1r[...], w1r[...], preferred_element_type=jnp.float32)
        acc = acc + jnp.dot(x2r[...], w2r[...], preferred_element_type=jnp.float32)
        acc = acc + jnp.dot(x3r[...], w3r[...], preferred_element_type=jnp.float32)
        acc = acc * sr[...] + br[...]
        if relu:
            acc = jnp.maximum(acc, 0.0)
        o[...] = acc

    in_specs = (
        [pl.BlockSpec((bn, f), lambda i: (i, 0))] * 4
        + [pl.BlockSpec((f, fo), lambda i: (0, 0))] * 4
        + [pl.BlockSpec((1, fo), lambda i: (0, 0))] * 2
    )
    return pl.pallas_call(
        body,
        grid=(n // bn,),
        in_specs=in_specs,
        out_specs=pl.BlockSpec((bn, fo), lambda i: (i, 0)),
        out_shape=jax.ShapeDtypeStruct((n, fo), jnp.float32),
    )(x0, x1, x2, x3, w0, w1, w2, w3, svec, bvec)


def kernel(x, edge_index, edge_weight, W1, b1, gamma, beta, W2, b2):
    n, f = x.shape
    e = edge_index.shape[1]
    nw = _NC * _NS
    ch = -(-e // (nw * _C * 6)) * 6
    epad = nw * ch * _C - e

    src = edge_index[0].astype(jnp.int32)
    dst = edge_index[1].astype(jnp.int32)
    w = edge_weight.astype(jnp.float32)
    if epad:
        src = jnp.concatenate([src, jnp.zeros((epad,), jnp.int32)])
        dst = jnp.concatenate([dst, jnp.zeros((epad,), jnp.int32)])
        w = jnp.concatenate([w, jnp.zeros((epad,), jnp.float32)])

    def prop(hcur):
        p = _propagate_sc(hcur, src, dst, w, ch)
        return p[0] + p[1]

    ax = prop(x)
    a2x = prop(ax)
    a3x = prop(a2x)
    s = gamma * jax.lax.rsqrt(jnp.float32(1.0 + 1e-3))
    bvec = (b1 * s + beta).reshape(1, -1)
    h = _mm_fused(x, ax, a2x, a3x,
                  W1[0:f], W1[f:2 * f], W1[2 * f:3 * f], W1[3 * f:4 * f],
                  s.reshape(1, -1), bvec, relu=True)

    ah = prop(h)
    a2h = prop(ah)
    a3h = prop(a2h)
    fo = W2.shape[1]
    w2p = jnp.pad(W2, ((0, 0), (0, 128 - fo)))
    ones = jnp.ones((1, 128), jnp.float32)
    b2p = jnp.pad(b2, (0, 128 - fo)).reshape(1, -1)
    h2 = h.shape[1]
    z = _mm_fused(h, ah, a2h, a3h,
                  w2p[0:h2], w2p[h2:2 * h2], w2p[2 * h2:3 * h2], w2p[3 * h2:4 * h2],
                  ones, b2p, relu=False)
    return z[:, :fo]
